# baseline (speedup 1.0000x reference)
import dataclasses
import functools

import jax
import jax.numpy as jnp
from jax import lax
from jax.experimental import pallas as pl
from jax.experimental.pallas import tpu as pltpu
from jax.experimental.pallas import tpu_sc as plsc

NUM_CORES = 2
NUM_SUBCORES = 16
NW = NUM_CORES * NUM_SUBCORES
L = 16
D = 64
B = 16384
BPW = B // NW
CH = 128
CW = 2560
NBLK = 196
NROWS = NBLK * CW
OFFBLK = 195
OFF = OFFBLK * CW


def _tc_transpose_body(x1_ref, x2_ref, o_ref):
    eye = (lax.broadcasted_iota(jnp.int32, (D, D), 0) ==
           lax.broadcasted_iota(jnp.int32, (D, D), 1)).astype(jnp.float32)
    dn = (((0,), (0,)), ((), ()))
    t1 = lax.dot_general(x1_ref[...], eye, dn,
                         precision=lax.Precision.HIGHEST)
    t2 = lax.dot_general(x2_ref[...], eye, dn,
                         precision=lax.Precision.HIGHEST)
    o_ref[...] = jnp.concatenate([t1, t2], axis=1)


def _tc_transpose(emb_t):
    return pl.pallas_call(
        _tc_transpose_body,
        out_shape=jax.ShapeDtypeStruct((NROWS, 2 * D), jnp.float32),
        grid=(NBLK,),
        in_specs=[
            pl.BlockSpec((D, CW), lambda p: (0, p)),
            pl.BlockSpec((D, CW), lambda p: (0, p + OFFBLK)),
        ],
        out_specs=pl.BlockSpec((CW, 2 * D), lambda p: (p, 0)),
    )(emb_t, emb_t)


def _sc_bias_body(uid_hbm, iid_hbm, ubr_hbm, ibr_hbm, out_hbm,
                  uid_v, iid_v, ubi_v, ibi_v, ub_v, ib_v, o_v, sem0, sem1):
    wid = lax.axis_index("s") * NUM_CORES + lax.axis_index("c")
    base = wid * BPW

    pltpu.sync_copy(uid_hbm.at[pl.ds(base, BPW)], uid_v)
    pltpu.sync_copy(iid_hbm.at[pl.ds(base, BPW)], iid_v)

    @pl.loop(0, BPW, step=L)
    def _(o):
        ubi_v[pl.ds(o, L)] = lax.shift_right_logical(uid_v[pl.ds(o, L)], 4)
        ibi_v[pl.ds(o, L)] = lax.shift_right_logical(iid_v[pl.ds(o, L)], 4)

    cub = pltpu.async_copy(ubr_hbm.at[ubi_v], ub_v, sem0)
    cib = pltpu.async_copy(ibr_hbm.at[ibi_v], ib_v, sem1)
    cub.wait()
    cib.wait()

    iota = lax.iota(jnp.int32, L)
    fifteen = jnp.full((L,), 15, jnp.int32)

    @pl.loop(0, BPW, step=L)
    def _(g):
        ul = lax.bitwise_and(uid_v[pl.ds(g, L)], fifteen)
        il = lax.bitwise_and(iid_v[pl.ds(g, L)], fifteen)
        o_v[pl.ds(g, L)] = (plsc.load_gather(ub_v, [g + iota, ul]) +
                            plsc.load_gather(ib_v, [g + iota, il]))

    pltpu.sync_copy(o_v, out_hbm.at[pl.ds(base, BPW)])


def _sc_pair_body(uid_hbm, iid_hbm, up_hbm, ip_hbm, bsum_hbm, out_hbm,
                  uid_v, iid_v, urid_v, irid_v, u0_v, u1_v, i0_v, i1_v,
                  bs_v, o_v, a_ll, a_lh, a_hl, a_hh,
                  us0, us1, is0, is1):
    wid = lax.axis_index("s") * NUM_CORES + lax.axis_index("c")
    base = wid * BPW

    pltpu.sync_copy(uid_hbm.at[pl.ds(base, BPW)], uid_v)
    pltpu.sync_copy(iid_hbm.at[pl.ds(base, BPW)], iid_v)
    pltpu.sync_copy(bsum_hbm.at[pl.ds(base, BPW)], bs_v)

    half = jnp.full((L,), NROWS, jnp.int32)
    off = jnp.full((L,), OFF, jnp.int32)
    zero = jnp.zeros((L,), jnp.int32)

    @pl.loop(0, BPW, step=L)
    def _(o):
        u = uid_v[pl.ds(o, L)]
        i = iid_v[pl.ds(o, L)]
        urid_v[pl.ds(o, L)] = u - jnp.where(u >= half, off, zero)
        irid_v[pl.ds(o, L)] = i - jnp.where(i >= half, off, zero)

    u_bufs = (u0_v, u1_v)
    i_bufs = (i0_v, i1_v)
    usems = (us0, us1)
    isems = (is0, is1)

    pending = {}

    def issue(buf, c):
        cu = pltpu.async_copy(up_hbm.at[urid_v.at[pl.ds(c * CH, CH)]],
                              u_bufs[buf], usems[buf])
        ci = pltpu.async_copy(ip_hbm.at[irid_v.at[pl.ds(c * CH, CH)]],
                              i_bufs[buf], isems[buf])
        pending[buf] = (cu, ci)

    iota = lax.iota(jnp.int32, L)
    accs = (a_ll, a_lh, a_hl, a_hh)

    def process(buf, c):
        cu, ci = pending.pop(buf)
        cu.wait()
        ci.wait()
        ub = u_bufs[buf]
        ib = i_bufs[buf]

        @pl.loop(0, CH, step=L)
        def _(g):
            for j in range(L):
                r = g + j
                ulo = [ub[r, pl.ds(k, L)] for k in range(0, D, L)]
                uhi = [ub[r, pl.ds(D + k, L)] for k in range(0, D, L)]
                ilo = [ib[r, pl.ds(k, L)] for k in range(0, D, L)]
                ihi = [ib[r, pl.ds(D + k, L)] for k in range(0, D, L)]

                def dot4(a, b):
                    s = a[0] * b[0]
                    for q in range(1, 4):
                        s = s + a[q] * b[q]
                    return s

                a_ll[j, pl.ds(0, L)] = dot4(ulo, ilo)
                a_lh[j, pl.ds(0, L)] = dot4(ulo, ihi)
                a_hl[j, pl.ds(0, L)] = dot4(uhi, ilo)
                a_hh[j, pl.ds(0, L)] = dot4(uhi, ihi)

            tots = []
            for acc in accs:
                t = plsc.load_gather(acc, [iota, jnp.zeros((L,), jnp.int32)])
                for k in range(1, L):
                    t = t + plsc.load_gather(
                        acc, [iota, jnp.full((L,), k, jnp.int32)])
                tots.append(t)

            gg = c * CH + g
            mu = uid_v[pl.ds(gg, L)] >= half
            mi = iid_v[pl.ds(gg, L)] >= half
            lo = jnp.where(mi, tots[1], tots[0])
            hi = jnp.where(mi, tots[3], tots[2])
            o_v[pl.ds(gg, L)] = jnp.where(mu, hi, lo) + bs_v[pl.ds(gg, L)]

    issue(0, 0)
    issue(1, 1)
    process(0, 0)
    issue(0, 2)
    process(1, 1)
    issue(1, 3)
    process(0, 2)
    process(1, 3)

    pltpu.sync_copy(o_v, out_hbm.at[pl.ds(base, BPW)])


def _sc_compiler_params(use_tc_tiling):
    cp = pltpu.CompilerParams()
    if "needs_layout_passes" in pltpu.CompilerParams.__dataclass_fields__:
        cp = dataclasses.replace(cp, needs_layout_passes=False)
    if "use_tc_tiling_on_sc" in pltpu.CompilerParams.__dataclass_fields__:
        cp = dataclasses.replace(cp, use_tc_tiling_on_sc=use_tc_tiling)
    return cp


def kernel(user_ids, item_ids, user_emb, item_emb, user_bias, item_bias):
    uid = user_ids.astype(jnp.int32)
    iid = item_ids.astype(jnp.int32)
    nu = user_bias.shape[0]
    ni = item_bias.shape[0]
    ubias_rows = user_bias.reshape(nu // L, L)
    ibias_rows = item_bias.reshape(ni // L, L)

    upair = _tc_transpose(user_emb.T)
    ipair = _tc_transpose(item_emb.T)

    mesh = plsc.VectorSubcoreMesh(core_axis_name="c", subcore_axis_name="s",
                                  num_cores=NUM_CORES,
                                  num_subcores=NUM_SUBCORES)

    bias_call = pl.kernel(
        _sc_bias_body,
        out_type=jax.ShapeDtypeStruct((B,), jnp.float32),
        mesh=mesh,
        scratch_types=[
            pltpu.VMEM((BPW,), jnp.int32),
            pltpu.VMEM((BPW,), jnp.int32),
            pltpu.VMEM((BPW,), jnp.int32),
            pltpu.VMEM((BPW,), jnp.int32),
            pltpu.VMEM((BPW, L), jnp.float32),
            pltpu.VMEM((BPW, L), jnp.float32),
            pltpu.VMEM((BPW,), jnp.float32),
            pltpu.SemaphoreType.DMA,
            pltpu.SemaphoreType.DMA,
        ],
        compiler_params=_sc_compiler_params(False),
    )
    bsum = bias_call(uid, iid, ubias_rows, ibias_rows)

    pair_call = pl.kernel(
        _sc_pair_body,
        out_type=jax.ShapeDtypeStruct((B,), jnp.float32),
        mesh=mesh,
        scratch_types=[
            pltpu.VMEM((BPW,), jnp.int32),
            pltpu.VMEM((BPW,), jnp.int32),
            pltpu.VMEM((BPW,), jnp.int32),
            pltpu.VMEM((BPW,), jnp.int32),
            pltpu.VMEM((CH, 2 * D), jnp.float32),
            pltpu.VMEM((CH, 2 * D), jnp.float32),
            pltpu.VMEM((CH, 2 * D), jnp.float32),
            pltpu.VMEM((CH, 2 * D), jnp.float32),
            pltpu.VMEM((BPW,), jnp.float32),
            pltpu.VMEM((BPW,), jnp.float32),
            pltpu.VMEM((L, L), jnp.float32),
            pltpu.VMEM((L, L), jnp.float32),
            pltpu.VMEM((L, L), jnp.float32),
            pltpu.VMEM((L, L), jnp.float32),
            pltpu.SemaphoreType.DMA,
            pltpu.SemaphoreType.DMA,
            pltpu.SemaphoreType.DMA,
            pltpu.SemaphoreType.DMA,
        ],
        compiler_params=_sc_compiler_params(True),
    )
    return pair_call(uid, iid, upair, ipair, bsum)


# speedup vs baseline: 1.6204x; 1.6204x over previous
import dataclasses
import functools

import jax
import jax.numpy as jnp
from jax import lax
from jax.experimental import pallas as pl
from jax.experimental.pallas import tpu as pltpu
from jax.experimental.pallas import tpu_sc as plsc

NUM_CORES = 2
NUM_SUBCORES = 16
NW = NUM_CORES * NUM_SUBCORES
L = 16
D = 64
B = 16384
BPW = B // NW
CH = 128
CW = 2560
NBLK = 196
NROWS = NBLK * CW
OFFBLK = 195
OFF = OFFBLK * CW


def _tc_transpose_body(x1_ref, x2_ref, o_ref):
    t1 = jnp.transpose(x1_ref[...])
    t2 = jnp.transpose(x2_ref[...])
    o_ref[...] = jnp.concatenate([t1, t2], axis=1)


def _tc_transpose(emb_t):
    return pl.pallas_call(
        _tc_transpose_body,
        out_shape=jax.ShapeDtypeStruct((NROWS, 2 * D), jnp.float32),
        grid=(NBLK,),
        in_specs=[
            pl.BlockSpec((D, CW), lambda p: (0, p)),
            pl.BlockSpec((D, CW), lambda p: (0, p + OFFBLK)),
        ],
        out_specs=pl.BlockSpec((CW, 2 * D), lambda p: (p, 0)),
    )(emb_t, emb_t)


def _sc_bias_body(uid_hbm, iid_hbm, ubr_hbm, ibr_hbm, out_hbm,
                  uid_v, iid_v, ubi_v, ibi_v, ub_v, ib_v, o_v, sem0, sem1):
    wid = lax.axis_index("s") * NUM_CORES + lax.axis_index("c")
    base = wid * BPW

    pltpu.sync_copy(uid_hbm.at[pl.ds(base, BPW)], uid_v)
    pltpu.sync_copy(iid_hbm.at[pl.ds(base, BPW)], iid_v)

    @pl.loop(0, BPW, step=L)
    def _(o):
        ubi_v[pl.ds(o, L)] = lax.shift_right_logical(uid_v[pl.ds(o, L)], 4)
        ibi_v[pl.ds(o, L)] = lax.shift_right_logical(iid_v[pl.ds(o, L)], 4)

    cub = pltpu.async_copy(ubr_hbm.at[ubi_v], ub_v, sem0)
    cib = pltpu.async_copy(ibr_hbm.at[ibi_v], ib_v, sem1)
    cub.wait()
    cib.wait()

    iota = lax.iota(jnp.int32, L)
    fifteen = jnp.full((L,), 15, jnp.int32)

    @pl.loop(0, BPW, step=L)
    def _(g):
        ul = lax.bitwise_and(uid_v[pl.ds(g, L)], fifteen)
        il = lax.bitwise_and(iid_v[pl.ds(g, L)], fifteen)
        o_v[pl.ds(g, L)] = (plsc.load_gather(ub_v, [g + iota, ul]) +
                            plsc.load_gather(ib_v, [g + iota, il]))

    pltpu.sync_copy(o_v, out_hbm.at[pl.ds(base, BPW)])


def _sc_pair_body(uid_hbm, iid_hbm, up_hbm, ip_hbm, bsum_hbm, out_hbm,
                  uid_v, iid_v, urid_v, irid_v, u0_v, u1_v, i0_v, i1_v,
                  bs_v, o_v, a_ll, a_lh, a_hl, a_hh,
                  us0, us1, is0, is1):
    wid = lax.axis_index("s") * NUM_CORES + lax.axis_index("c")
    base = wid * BPW

    pltpu.sync_copy(uid_hbm.at[pl.ds(base, BPW)], uid_v)
    pltpu.sync_copy(iid_hbm.at[pl.ds(base, BPW)], iid_v)
    pltpu.sync_copy(bsum_hbm.at[pl.ds(base, BPW)], bs_v)

    half = jnp.full((L,), NROWS, jnp.int32)
    off = jnp.full((L,), OFF, jnp.int32)
    zero = jnp.zeros((L,), jnp.int32)

    @pl.loop(0, BPW, step=L)
    def _(o):
        u = uid_v[pl.ds(o, L)]
        i = iid_v[pl.ds(o, L)]
        urid_v[pl.ds(o, L)] = u - jnp.where(u >= half, off, zero)
        irid_v[pl.ds(o, L)] = i - jnp.where(i >= half, off, zero)

    u_bufs = (u0_v, u1_v)
    i_bufs = (i0_v, i1_v)
    usems = (us0, us1)
    isems = (is0, is1)

    pending = {}

    def issue(buf, c):
        cu = pltpu.async_copy(up_hbm.at[urid_v.at[pl.ds(c * CH, CH)]],
                              u_bufs[buf], usems[buf])
        ci = pltpu.async_copy(ip_hbm.at[irid_v.at[pl.ds(c * CH, CH)]],
                              i_bufs[buf], isems[buf])
        pending[buf] = (cu, ci)

    iota = lax.iota(jnp.int32, L)
    accs = (a_ll, a_lh, a_hl, a_hh)

    def process(buf, c):
        cu, ci = pending.pop(buf)
        cu.wait()
        ci.wait()
        ub = u_bufs[buf]
        ib = i_bufs[buf]

        @pl.loop(0, CH, step=L)
        def _(g):
            for j in range(L):
                r = g + j
                ulo = [ub[r, pl.ds(k, L)] for k in range(0, D, L)]
                uhi = [ub[r, pl.ds(D + k, L)] for k in range(0, D, L)]
                ilo = [ib[r, pl.ds(k, L)] for k in range(0, D, L)]
                ihi = [ib[r, pl.ds(D + k, L)] for k in range(0, D, L)]

                def dot4(a, b):
                    s = a[0] * b[0]
                    for q in range(1, 4):
                        s = s + a[q] * b[q]
                    return s

                a_ll[j, pl.ds(0, L)] = dot4(ulo, ilo)
                a_lh[j, pl.ds(0, L)] = dot4(ulo, ihi)
                a_hl[j, pl.ds(0, L)] = dot4(uhi, ilo)
                a_hh[j, pl.ds(0, L)] = dot4(uhi, ihi)

            tots = []
            for acc in accs:
                t = plsc.load_gather(acc, [iota, jnp.zeros((L,), jnp.int32)])
                for k in range(1, L):
                    t = t + plsc.load_gather(
                        acc, [iota, jnp.full((L,), k, jnp.int32)])
                tots.append(t)

            gg = c * CH + g
            mu = uid_v[pl.ds(gg, L)] >= half
            mi = iid_v[pl.ds(gg, L)] >= half
            lo = jnp.where(mi, tots[1], tots[0])
            hi = jnp.where(mi, tots[3], tots[2])
            o_v[pl.ds(gg, L)] = jnp.where(mu, hi, lo) + bs_v[pl.ds(gg, L)]

    issue(0, 0)
    issue(1, 1)
    process(0, 0)
    issue(0, 2)
    process(1, 1)
    issue(1, 3)
    process(0, 2)
    process(1, 3)

    pltpu.sync_copy(o_v, out_hbm.at[pl.ds(base, BPW)])


def _sc_compiler_params(use_tc_tiling):
    cp = pltpu.CompilerParams()
    if "needs_layout_passes" in pltpu.CompilerParams.__dataclass_fields__:
        cp = dataclasses.replace(cp, needs_layout_passes=False)
    if "use_tc_tiling_on_sc" in pltpu.CompilerParams.__dataclass_fields__:
        cp = dataclasses.replace(cp, use_tc_tiling_on_sc=use_tc_tiling)
    return cp


def kernel(user_ids, item_ids, user_emb, item_emb, user_bias, item_bias):
    uid = user_ids.astype(jnp.int32)
    iid = item_ids.astype(jnp.int32)
    nu = user_bias.shape[0]
    ni = item_bias.shape[0]
    ubias_rows = user_bias.reshape(nu // L, L)
    ibias_rows = item_bias.reshape(ni // L, L)

    upair = _tc_transpose(user_emb.T)
    ipair = _tc_transpose(item_emb.T)

    mesh = plsc.VectorSubcoreMesh(core_axis_name="c", subcore_axis_name="s",
                                  num_cores=NUM_CORES,
                                  num_subcores=NUM_SUBCORES)

    bias_call = pl.kernel(
        _sc_bias_body,
        out_type=jax.ShapeDtypeStruct((B,), jnp.float32),
        mesh=mesh,
        scratch_types=[
            pltpu.VMEM((BPW,), jnp.int32),
            pltpu.VMEM((BPW,), jnp.int32),
            pltpu.VMEM((BPW,), jnp.int32),
            pltpu.VMEM((BPW,), jnp.int32),
            pltpu.VMEM((BPW, L), jnp.float32),
            pltpu.VMEM((BPW, L), jnp.float32),
            pltpu.VMEM((BPW,), jnp.float32),
            pltpu.SemaphoreType.DMA,
            pltpu.SemaphoreType.DMA,
        ],
        compiler_params=_sc_compiler_params(False),
    )
    bsum = bias_call(uid, iid, ubias_rows, ibias_rows)

    pair_call = pl.kernel(
        _sc_pair_body,
        out_type=jax.ShapeDtypeStruct((B,), jnp.float32),
        mesh=mesh,
        scratch_types=[
            pltpu.VMEM((BPW,), jnp.int32),
            pltpu.VMEM((BPW,), jnp.int32),
            pltpu.VMEM((BPW,), jnp.int32),
            pltpu.VMEM((BPW,), jnp.int32),
            pltpu.VMEM((CH, 2 * D), jnp.float32),
            pltpu.VMEM((CH, 2 * D), jnp.float32),
            pltpu.VMEM((CH, 2 * D), jnp.float32),
            pltpu.VMEM((CH, 2 * D), jnp.float32),
            pltpu.VMEM((BPW,), jnp.float32),
            pltpu.VMEM((BPW,), jnp.float32),
            pltpu.VMEM((L, L), jnp.float32),
            pltpu.VMEM((L, L), jnp.float32),
            pltpu.VMEM((L, L), jnp.float32),
            pltpu.VMEM((L, L), jnp.float32),
            pltpu.SemaphoreType.DMA,
            pltpu.SemaphoreType.DMA,
            pltpu.SemaphoreType.DMA,
            pltpu.SemaphoreType.DMA,
        ],
        compiler_params=_sc_compiler_params(True),
    )
    return pair_call(uid, iid, upair, ipair, bsum)
